# Initial kernel scaffold; baseline (speedup 1.0000x reference)
import jax
import jax.numpy as jnp
from jax.experimental import pallas as pl


def kernel(w1a, w1b, w2a, w2b, g1a, b1a, g1b, b1b, g2a, b2a, g2b, b2b, wlin, blin, x, skip_x, embeddings):
    raise NotImplementedError("write your pallas kernel here")


# speedup vs baseline: 1.0643x; 1.0643x over previous
import functools

import jax
import jax.numpy as jnp
from jax.experimental import pallas as pl
from jax.experimental.pallas import tpu as pltpu

_EPS = 1e-5
_INV_SQRT2 = 0.7071067811865476


def _gelu(v):
    return 0.5 * v * (1.0 + jax.lax.erf(v * _INV_SQRT2))


def _batch_block(N, cap=8):
    best = 1
    for d in range(2, min(N, cap) + 1):
        if N % d == 0 and (N // d) >= 2:
            best = d
    return best


def _make_body(Nb, H, W, Cin, Cmid, Cout):
    HW = H * W
    M = Nb * HW

    def body(x_ref, w1a_ref, w1b_ref, w2a_ref, w2b_ref,
             g1a_ref, b1a_ref, g1b_ref, b1b_ref,
             g2a_ref, b2a_ref, g2b_ref, b2b_ref,
             emb_ref, out_ref):
        row = jax.lax.broadcasted_iota(jnp.int32, (M, 1), 0)
        wc = jax.lax.rem(row, W)
        hc = jax.lax.rem(jax.lax.div(row, W), H)
        one = jnp.float32(1.0)
        mask_wlo = (wc >= 1).astype(jnp.bfloat16)
        mask_whi = (wc <= W - 2).astype(jnp.bfloat16)
        mask_hlo = (hc >= 1).astype(jnp.bfloat16)
        mask_hhi = (hc <= H - 2).astype(jnp.bfloat16)
        del one

        def conv3x3(act, w_ref):
            Ci = act.shape[1]
            a0 = act.astype(jnp.bfloat16)
            zrow = jnp.zeros((1, Ci), jnp.bfloat16)
            am = jnp.concatenate([zrow, a0[:-1, :]], axis=0) * mask_wlo
            ap = jnp.concatenate([a0[1:, :], zrow], axis=0) * mask_whi
            zpad = jnp.zeros((W, Ci), jnp.bfloat16)
            exts = [jnp.concatenate([zpad, a, zpad], axis=0)
                    for a in (am, a0, ap)]
            pieces = []
            for dy in range(3):
                for dx in range(3):
                    p = exts[dx][dy * W:dy * W + M, :]
                    if dy == 0:
                        p = p * mask_hlo
                    elif dy == 2:
                        p = p * mask_hhi
                    pieces.append(p)
            col = jnp.concatenate(pieces, axis=1)
            return jnp.dot(col, w_ref[...],
                           preferred_element_type=jnp.float32)

        def group_norm(h, g_ref, b_ref, add_emb=False):
            gamma = g_ref[...]
            beta = b_ref[...]
            inv_n = 1.0 / float(h.shape[1] * HW)
            pieces = []
            for b in range(Nb):
                blk = h[b * HW:(b + 1) * HW, :]
                mean = jnp.sum(blk, keepdims=True) * inv_n
                cent = blk - mean
                var = jnp.sum(cent * cent, keepdims=True) * inv_n
                y = cent * jax.lax.rsqrt(var + _EPS) * gamma + beta
                if add_emb:
                    y = y + emb_ref[b:b + 1, :]
                pieces.append(y)
            return pieces[0] if Nb == 1 else jnp.concatenate(pieces, axis=0)

        x = x_ref[...]

        h = conv3x3(x, w1a_ref)
        h = _gelu(group_norm(h, g1a_ref, b1a_ref))
        h = conv3x3(h, w1b_ref)
        h = group_norm(h, g1b_ref, b1b_ref)
        r = _gelu(x + h)

        m = conv3x3(r, w2a_ref)
        m = _gelu(group_norm(m, g2a_ref, b2a_ref))
        o = conv3x3(m, w2b_ref)
        o = group_norm(o, g2b_ref, b2b_ref, add_emb=True)

        out_ref[...] = o.astype(out_ref.dtype)

    return body


def _upsample2x(x):
    N, C, Hin, Win = x.shape
    Hout, Wout = 2 * Hin, 2 * Win

    def coords(n_in, n_out):
        src = jnp.arange(n_out, dtype=jnp.float32) * (n_in - 1) / (n_out - 1)
        lo = jnp.clip(jnp.floor(src).astype(jnp.int32), 0, n_in - 2)
        frac = src - lo.astype(jnp.float32)
        return lo, lo + 1, frac

    hlo, hhi, fh = coords(Hin, Hout)
    wlo, whi, fw = coords(Win, Wout)
    top = (x[:, :, hlo, :] * (1.0 - fh)[None, None, :, None]
           + x[:, :, hhi, :] * fh[None, None, :, None])
    return (top[:, :, :, wlo] * (1.0 - fw)[None, None, None, :]
            + top[:, :, :, whi] * fw[None, None, None, :])


@jax.jit
def kernel(w1a, w1b, w2a, w2b, g1a, b1a, g1b, b1b,
           g2a, b2a, g2b, b2b, wlin, blin, x, skip_x, embeddings):
    xu = _upsample2x(x)
    xc = jnp.concatenate([skip_x, xu], axis=1)
    N, Cin, H, W = xc.shape
    HW = H * W

    xf = jnp.transpose(xc, (0, 2, 3, 1)).reshape(N * HW, Cin)
    xf = xf.astype(jnp.float32)

    Cmid = w2a.shape[-1]
    Cout = w2b.shape[-1]

    wb1a = w1a.reshape(9 * Cin, Cin).astype(jnp.bfloat16)
    wb1b = w1b.reshape(9 * Cin, Cin).astype(jnp.bfloat16)
    wb2a = w2a.reshape(9 * Cin, Cmid).astype(jnp.bfloat16)
    wb2b = w2b.reshape(9 * Cmid, Cout).astype(jnp.bfloat16)

    ga1a = g1a.reshape(1, Cin)
    bb1a = b1a.reshape(1, Cin)
    ga1b = g1b.reshape(1, Cin)
    bb1b = b1b.reshape(1, Cin)
    ga2a = g2a.reshape(1, Cmid)
    bb2a = b2a.reshape(1, Cmid)
    ga2b = g2b.reshape(1, Cout)
    bb2b = b2b.reshape(1, Cout)

    e = embeddings.astype(jnp.float32)
    e = e * jax.nn.sigmoid(e)
    emb = e @ wlin + blin

    Nb = _batch_block(N)
    M = Nb * HW
    body = _make_body(Nb, H, W, Cin, Cmid, Cout)

    def full(a):
        nd = a.ndim
        return pl.BlockSpec(a.shape, lambda b: (0,) * nd)

    out = pl.pallas_call(
        body,
        out_shape=jax.ShapeDtypeStruct((N * HW, Cout), jnp.float32),
        grid=(N // Nb,),
        in_specs=[
            pl.BlockSpec((M, Cin), lambda b: (b, 0)),
            full(wb1a), full(wb1b), full(wb2a), full(wb2b),
            full(ga1a), full(bb1a), full(ga1b), full(bb1b),
            full(ga2a), full(bb2a), full(ga2b), full(bb2b),
            pl.BlockSpec((Nb, Cout), lambda b: (b, 0)),
        ],
        out_specs=pl.BlockSpec((M, Cout), lambda b: (b, 0)),
        compiler_params=pltpu.CompilerParams(
            dimension_semantics=("parallel",)),
    )(xf, wb1a, wb1b, wb2a, wb2b,
      ga1a, bb1a, ga1b, bb1b, ga2a, bb2a, ga2b, bb2b, emb)

    return jnp.transpose(out.reshape(N, HW, Cout), (0, 2, 1)).reshape(
        N, Cout, H, W)
